# baseline (speedup 1.0000x reference)
import functools

import jax
import jax.numpy as jnp
from jax import lax
from jax.experimental import pallas as pl
from jax.experimental.pallas import tpu as pltpu
from jax.experimental.pallas import tpu_sc as plsc

N_MATCH = 16384
N_HERO = 20
PAD = 448
NUM_WORKERS = 32
M_PER_W = N_MATCH // NUM_WORKERS
IDX_PER_W = M_PER_W * N_HERO
LANES = 16
N_CHUNKS = M_PER_W // LANES

_mesh = plsc.VectorSubcoreMesh(core_axis_name="c", subcore_axis_name="s")


@functools.partial(
    pl.kernel,
    out_type=jax.ShapeDtypeStruct((N_MATCH,), jnp.float32),
    mesh=_mesh,
    scratch_types=[
        pltpu.VMEM((IDX_PER_W,), jnp.int32),
        pltpu.VMEM((IDX_PER_W,), jnp.float32),
        pltpu.VMEM((M_PER_W,), jnp.float32),
        pltpu.SemaphoreType.DMA,
    ],
    compiler_params=pltpu.CompilerParams(needs_layout_passes=False),
)
def _team_sum(team_hbm, skill_hbm, out_hbm, idx_v, vals_v, acc_v, sem):
    wid = lax.axis_index("s") * 2 + lax.axis_index("c")
    mbase = wid * M_PER_W

    stages = [
        pltpu.async_copy(
            team_hbm.at[j, pl.ds(mbase, M_PER_W)],
            idx_v.at[pl.ds(j * M_PER_W, M_PER_W)],
            sem,
        )
        for j in range(N_HERO)
    ]
    for d in stages:
        d.wait()
    pltpu.async_copy(skill_hbm.at[idx_v], vals_v, sem).wait()

    def chunk_body(c, _):
        m16 = c * LANES
        vs = [vals_v[pl.ds(j * M_PER_W + m16, LANES)] for j in range(N_HERO)]
        while len(vs) > 1:
            vs = [a + b for a, b in zip(vs[::2], vs[1::2])] + vs[2 * (len(vs) // 2):]
        acc_v[pl.ds(m16, LANES)] = vs[0]
        return _

    lax.fori_loop(0, N_CHUNKS, chunk_body, None)
    pltpu.sync_copy(acc_v, out_hbm.at[pl.ds(mbase, M_PER_W)])


def kernel(team, skill):
    team_t = team.T.astype(jnp.int32)
    skill_flat = jnp.concatenate(
        [skill, jnp.zeros((PAD, 1), jnp.float32)]
    ).reshape(-1)
    out = _team_sum(team_t, skill_flat)
    return out.reshape(N_MATCH, 1)
